# baseline (speedup 1.0000x reference)
import functools

import jax
import jax.numpy as jnp
from jax import lax
from jax.experimental import pallas as pl
from jax.experimental.pallas import tpu as pltpu
from jax.experimental.pallas import tpu_sc as plsc

N = 10000
E = 160000
NROW = 10112
RPT = 632
CHK = 128
NCHK = 81
EPAD_A = 16 * NCHK * CHK
CH = 128
NCH_DEG = 40
EPAD_D = 32 * NCH_DEG * CH
NSLOT_A = 3
NSLOT = 4
D = 256
F32 = jnp.float32

@functools.lru_cache(maxsize=None)
def _mesh():
    return plsc.VectorSubcoreMesh(core_axis_name="c", subcore_axis_name="s",
                                  num_cores=2, num_subcores=16)


def _deg_body(dsts, zrows8, ones8, out, hist, ones_v, dbufs, ddsems):
    c = lax.axis_index("c")
    s = lax.axis_index("s")
    t = c * 16 + s
    pltpu.sync_copy(ones8, ones_v)
    r0 = s * RPT
    pltpu.sync_copy(zrows8.at[pl.ds(r0, RPT)], hist.at[pl.ds(r0, RPT)])

    def idx_cp(ci, b):
        return pltpu.async_copy(dsts.at[t, ci], dbufs[b], ddsems[b])

    for b in range(2):
        idx_cp(b, b)
    plsc.subcore_barrier()

    @pl.loop(0, NCH_DEG // NSLOT)
    def _(g):
        for u in range(NSLOT):
            ci = NSLOT * g + u
            b2 = (u + 2) % NSLOT

            @pl.when(ci + 2 < NCH_DEG)
            def _():
                idx_cp(ci + 2, b2)

            pltpu.make_async_copy(dsts.at[t, ci], dbufs[u], ddsems[u]).wait()
            pltpu.sync_copy(ones_v, hist.at[dbufs[u]], add=True)

    plsc.subcore_barrier()
    pltpu.sync_copy(hist.at[pl.ds(r0, RPT)], out.at[c, pl.ds(r0, RPT)])


@functools.lru_cache(maxsize=None)
def _deg():
    return pl.kernel(
        _deg_body,
        out_type=jax.ShapeDtypeStruct((2, NROW, 8), F32),
        mesh=_mesh(),
        scratch_types=[
            pltpu.VMEM_SHARED((NROW, 8), F32),
            pltpu.VMEM((CH, 8), F32),
            [pltpu.VMEM((CH,), jnp.int32)] * NSLOT,
            [pltpu.SemaphoreType.DMA] * NSLOT,
        ],
    )


def _agg_body(ycat, srcs, dsts, zrows, out,
              acc, rows, sbufs, dbufs, gsems, sisems, ddsems):
    c = lax.axis_index("c")
    s = lax.axis_index("s")
    r0 = s * RPT
    pltpu.sync_copy(zrows.at[pl.ds(r0, RPT)], acc.at[pl.ds(r0, RPT)])

    def sidx_cp(ci, b):
        return pltpu.async_copy(srcs.at[c, s, ci], sbufs[b], sisems[b])

    def didx_cp(ci, b):
        return pltpu.async_copy(dsts.at[s, ci], dbufs[b], ddsems[b])

    def gat_cp(b):
        return pltpu.async_copy(ycat.at[sbufs[b]], rows[b], gsems[b])

    for b in range(NSLOT_A):
        sidx_cp(b, b)
    for b in range(2):
        didx_cp(b, b)
        pltpu.make_async_copy(srcs.at[c, s, b], sbufs[b], sisems[b]).wait()
        gat_cp(b)
    plsc.subcore_barrier()

    @pl.loop(0, NCHK // NSLOT_A)
    def _(g):
        for u in range(NSLOT_A):
            ci = NSLOT_A * g + u
            b2 = (u + 2) % NSLOT_A

            @pl.when(ci + 2 < NCHK)
            def _():
                pltpu.make_async_copy(srcs.at[c, s, ci], sbufs[b2],
                                      sisems[b2]).wait()
                didx_cp(ci + 2, b2)
                gat_cp(b2)

            pltpu.make_async_copy(ycat.at[sbufs[u]], rows[u], gsems[u]).wait()

            @pl.when(ci + 3 < NCHK)
            def _():
                sidx_cp(ci + 3, u)

            pltpu.make_async_copy(dsts.at[s, ci], dbufs[u], ddsems[u]).wait()
            pltpu.sync_copy(rows[u], acc.at[dbufs[u]], add=True)

    plsc.subcore_barrier()
    pltpu.sync_copy(acc.at[pl.ds(r0, RPT)], out.at[c, pl.ds(r0, RPT)])


@functools.lru_cache(maxsize=None)
def _agg():
    return pl.kernel(
        _agg_body,
        out_type=jax.ShapeDtypeStruct((2, NROW, 128), F32),
        mesh=_mesh(),
        scratch_types=[
            pltpu.VMEM_SHARED((NROW, 128), F32),
            [pltpu.VMEM((CHK, 128), F32)] * NSLOT_A,
            [pltpu.VMEM((CHK,), jnp.int32)] * NSLOT_A,
            [pltpu.VMEM((CHK,), jnp.int32)] * NSLOT_A,
            [pltpu.SemaphoreType.DMA] * NSLOT_A,
            [pltpu.SemaphoreType.DMA] * NSLOT_A,
            [pltpu.SemaphoreType.DMA] * NSLOT_A,
        ],
    )


ROWT = 400
GRID = N // ROWT


def _mm1_body(x_ref, w_ref, o_ref):
    o_ref[...] = lax.dot_general(x_ref[...], w_ref[...],
                                 (((1,), (1,)), ((), ())),
                                 preferred_element_type=F32)


_mm1 = pl.pallas_call(
    _mm1_body,
    grid=(GRID,),
    in_specs=[pl.BlockSpec((ROWT, D), lambda i: (i, 0)),
              pl.BlockSpec((D, D), lambda i: (0, 0))],
    out_specs=pl.BlockSpec((ROWT, D), lambda i: (i, 0)),
    out_shape=jax.ShapeDtypeStruct((N, D), F32),
)


def _scale_body(xw_ref, h8_ref, y_ref, dv_ref):
    hist = h8_ref[0, :, 0] + h8_ref[1, :, 0]
    dv = lax.rsqrt(hist + 1.0)
    dv_ref[...] = dv[:, None]
    y = xw_ref[...] * dv[:, None]
    y_ref[0] = y[:, :128]
    y_ref[1] = y[:, 128:]


_scale = pl.pallas_call(
    _scale_body,
    grid=(GRID,),
    in_specs=[pl.BlockSpec((ROWT, D), lambda i: (i, 0)),
              pl.BlockSpec((2, ROWT, 8), lambda i: (0, i, 0))],
    out_specs=[pl.BlockSpec((2, ROWT, 128), lambda i: (0, i, 0)),
               pl.BlockSpec((ROWT, 1), lambda i: (i, 0))],
    out_shape=[jax.ShapeDtypeStruct((2, N, 128), F32),
               jax.ShapeDtypeStruct((N, 1), F32)],
)


def _bcos1_body(agg_ref, y1_ref, dv_ref, wb_ref, w2_ref, z_ref, s_ref, y2_ref):
    a = jnp.concatenate([agg_ref[0], agg_ref[1]], axis=1)
    yy = jnp.concatenate([y1_ref[0], y1_ref[1]], axis=1)
    dv = dv_ref[...]
    z = dv * (a + yy)
    z_ref[...] = z
    rn = jnp.maximum(jnp.sqrt(jnp.sum(z * z, axis=1, keepdims=True)), 1e-12)
    wb = wb_ref[...]
    wn = jnp.maximum(jnp.sqrt(jnp.sum(wb * wb, axis=1)), 1e-12)
    lin = lax.dot_general(z, wb, (((1,), (1,)), ((), ())),
                          preferred_element_type=F32)
    sc = jnp.maximum(lin / (rn * wn[None, :]), 1e-6)
    s_ref[...] = sc
    h = jnp.maximum(lin * sc, 0.0)
    hw = lax.dot_general(h, w2_ref[...], (((1,), (1,)), ((), ())),
                         preferred_element_type=F32)
    y2 = dv * hw
    y2_ref[0] = y2[:, :128]
    y2_ref[1] = y2[:, 128:]


_bcos1 = pl.pallas_call(
    _bcos1_body,
    grid=(GRID,),
    in_specs=[pl.BlockSpec((2, ROWT, 128), lambda i: (0, i, 0)),
              pl.BlockSpec((2, ROWT, 128), lambda i: (0, i, 0)),
              pl.BlockSpec((ROWT, 1), lambda i: (i, 0)),
              pl.BlockSpec((D, D), lambda i: (0, 0)),
              pl.BlockSpec((D, D), lambda i: (0, 0))],
    out_specs=[pl.BlockSpec((ROWT, D), lambda i: (i, 0)),
               pl.BlockSpec((ROWT, D), lambda i: (i, 0)),
               pl.BlockSpec((2, ROWT, 128), lambda i: (0, i, 0))],
    out_shape=[jax.ShapeDtypeStruct((N, D), F32),
               jax.ShapeDtypeStruct((N, D), F32),
               jax.ShapeDtypeStruct((2, N, 128), F32)],
)


def _bcos2_body(agg_ref, y2_ref, dv_ref, wb_ref, z_ref, s_ref, o_ref):
    a = jnp.concatenate([agg_ref[0], agg_ref[1]], axis=1)
    yy = jnp.concatenate([y2_ref[0], y2_ref[1]], axis=1)
    dv = dv_ref[...]
    z = dv * (a + yy)
    z_ref[...] = z
    rn = jnp.maximum(jnp.sqrt(jnp.sum(z * z, axis=1, keepdims=True)), 1e-12)
    wb = wb_ref[...]
    wn = jnp.maximum(jnp.sqrt(jnp.sum(wb * wb, axis=1)), 1e-12)
    lin = lax.dot_general(z, wb, (((1,), (1,)), ((), ())),
                          preferred_element_type=F32)
    sc = jnp.maximum(lin / (rn * wn[None, :]), 1e-6)
    s_ref[...] = sc
    o_ref[...] = lin * sc


_bcos2 = pl.pallas_call(
    _bcos2_body,
    grid=(GRID,),
    in_specs=[pl.BlockSpec((2, ROWT, 128), lambda i: (0, i, 0)),
              pl.BlockSpec((2, ROWT, 128), lambda i: (0, i, 0)),
              pl.BlockSpec((ROWT, 1), lambda i: (i, 0)),
              pl.BlockSpec((128, D), lambda i: (0, 0))],
    out_specs=[pl.BlockSpec((ROWT, D), lambda i: (i, 0)),
               pl.BlockSpec((ROWT, 128), lambda i: (i, 0)),
               pl.BlockSpec((ROWT, 128), lambda i: (i, 0))],
    out_shape=[jax.ShapeDtypeStruct((N, D), F32),
               jax.ShapeDtypeStruct((N, 128), F32),
               jax.ShapeDtypeStruct((N, 128), F32)],
)


def kernel(x, edge_index, W1, W2, Wb1, Wb2):
    src = edge_index[0].astype(jnp.int32)
    dst = edge_index[1].astype(jnp.int32)
    src_p = jnp.concatenate([src, jnp.zeros((EPAD_A - E,), jnp.int32)])
    dst_p = jnp.concatenate([dst, jnp.full((EPAD_A - E,), N, jnp.int32)])
    srcs2 = jnp.stack([src_p, src_p + N]).reshape(2, 16, NCHK, CHK)
    dst_agg = dst_p.reshape(16, NCHK, CHK)
    dst_deg = jnp.concatenate(
        [dst, jnp.full((EPAD_D - E,), N, jnp.int32)]).reshape(32, NCH_DEG, CH)
    zrows = jnp.zeros((NROW, 128), F32)
    zrows8 = jnp.zeros((NROW, 8), F32)
    ones8 = jnp.concatenate(
        [jnp.ones((CH, 1), F32), jnp.zeros((CH, 7), F32)], axis=1)

    hist8 = _deg()(dst_deg, zrows8, ones8)
    xw = _mm1(x, W1)
    y1, dinv = _scale(xw, hist8)
    agg1 = _agg()(y1.reshape(2 * N, 128), srcs2, dst_agg, zrows)
    z1, s1, y2 = _bcos1(agg1, y1, dinv, Wb1, W2)
    agg2 = _agg()(y2.reshape(2 * N, 128), srcs2, dst_agg, zrows)
    z2, s2, out = _bcos2(agg2, y2, dinv, Wb2)
    return (out, z1, z2, s1, s2)


# speedup vs baseline: 1.3384x; 1.3384x over previous
import functools

import jax
import jax.numpy as jnp
from jax import lax
from jax.experimental import pallas as pl
from jax.experimental.pallas import tpu as pltpu
from jax.experimental.pallas import tpu_sc as plsc

N = 10000
E = 160000
NROW = 10240
RPT = 640
CHK = 128
NCHK = 80
EPAD = 16 * NCHK * CHK
NCH_DEG = 40
NSLOT = 4
D = 256
F32 = jnp.float32

@functools.lru_cache(maxsize=None)
def _mesh():
    return plsc.VectorSubcoreMesh(core_axis_name="c", subcore_axis_name="s",
                                  num_cores=2, num_subcores=16)


def _deg_body(dsts, zrows8, ones8, out, hist, ones_v, dbufs, ddsems):
    c = lax.axis_index("c")
    s = lax.axis_index("s")
    t = c * 16 + s
    pltpu.sync_copy(ones8, ones_v)
    r0 = s * RPT
    pltpu.sync_copy(zrows8.at[pl.ds(r0, RPT)], hist.at[pl.ds(r0, RPT)])

    def idx_cp(ci, b):
        return pltpu.async_copy(dsts.at[t, ci], dbufs[b], ddsems[b])

    for b in range(2):
        idx_cp(b, b)
    plsc.subcore_barrier()

    @pl.loop(0, NCH_DEG // NSLOT)
    def _(g):
        for u in range(NSLOT):
            ci = NSLOT * g + u
            b2 = (u + 2) % NSLOT

            @pl.when(ci + 2 < NCH_DEG)
            def _():
                idx_cp(ci + 2, b2)

            pltpu.make_async_copy(dsts.at[t, ci], dbufs[u], ddsems[u]).wait()
            pltpu.sync_copy(ones_v, hist.at[dbufs[u]], add=True)

    plsc.subcore_barrier()
    pltpu.sync_copy(hist.at[pl.ds(r0, RPT)], out.at[c, pl.ds(r0, RPT)])


@functools.lru_cache(maxsize=None)
def _deg():
    return pl.kernel(
        _deg_body,
        out_type=jax.ShapeDtypeStruct((2, NROW, 8), F32),
        mesh=_mesh(),
        scratch_types=[
            pltpu.VMEM_SHARED((NROW, 8), F32),
            pltpu.VMEM((CHK, 8), F32),
            [pltpu.VMEM((CHK,), jnp.int32)] * NSLOT,
            [pltpu.SemaphoreType.DMA] * NSLOT,
        ],
    )


def _agg_body(ycat, srcs, dsts, zrows, out,
              acc, srcmat, dbufs, rows, gsems, dsems):
    c = lax.axis_index("c")
    s = lax.axis_index("s")
    pltpu.sync_copy(srcs.at[c, s], srcmat)
    r0 = s * RPT
    pltpu.sync_copy(zrows.at[pl.ds(r0, RPT)], acc.at[pl.ds(r0, RPT)])
    plsc.subcore_barrier()

    for b in range(2):
        pltpu.async_copy(dsts.at[s, b], dbufs[b], dsems[b])
        pltpu.async_copy(ycat.at[srcmat.at[b]], rows[b], gsems[b])

    @pl.loop(0, NCHK // 2)
    def _(g):
        for b in range(2):
            ci = 2 * g + b
            pltpu.make_async_copy(ycat.at[srcmat.at[ci]], rows[b],
                                  gsems[b]).wait()
            pltpu.make_async_copy(dsts.at[s, ci], dbufs[b], dsems[b]).wait()
            pltpu.sync_copy(rows[b], acc.at[dbufs[b]], add=True)
            nxt = ci + 2

            @pl.when(nxt < NCHK)
            def _():
                pltpu.async_copy(dsts.at[s, nxt], dbufs[b], dsems[b])
                pltpu.async_copy(ycat.at[srcmat.at[nxt]], rows[b], gsems[b])

    plsc.subcore_barrier()
    pltpu.sync_copy(acc.at[pl.ds(r0, RPT)], out.at[c, pl.ds(r0, RPT)])


@functools.lru_cache(maxsize=None)
def _agg():
    return pl.kernel(
        _agg_body,
        out_type=jax.ShapeDtypeStruct((2, NROW, 128), F32),
        mesh=_mesh(),
        scratch_types=[
            pltpu.VMEM_SHARED((NROW, 128), F32),
            pltpu.VMEM((NCHK, CHK), jnp.int32),
            [pltpu.VMEM((CHK,), jnp.int32)] * 2,
            [pltpu.VMEM((CHK, 128), F32)] * 2,
            [pltpu.SemaphoreType.DMA] * 2,
            [pltpu.SemaphoreType.DMA] * 2,
        ],
    )


ROWT = 400
GRID = N // ROWT


def _mm1s_body(x_ref, w_ref, h_ref, y_ref, dv_ref):
    xw = lax.dot_general(x_ref[...], w_ref[...],
                         (((1,), (1,)), ((), ())),
                         preferred_element_type=F32)
    hist = h_ref[0, :, 0] + h_ref[1, :, 0]
    dv = lax.rsqrt(hist + 1.0)
    dv_ref[...] = dv[:, None]
    y = xw * dv[:, None]
    y_ref[0] = y[:, :128]
    y_ref[1] = y[:, 128:]


_mm1s = pl.pallas_call(
    _mm1s_body,
    grid=(GRID,),
    in_specs=[pl.BlockSpec((ROWT, D), lambda i: (i, 0)),
              pl.BlockSpec((D, D), lambda i: (0, 0)),
              pl.BlockSpec((2, ROWT, 8), lambda i: (0, i, 0))],
    out_specs=[pl.BlockSpec((2, ROWT, 128), lambda i: (0, i, 0)),
               pl.BlockSpec((ROWT, 1), lambda i: (i, 0))],
    out_shape=[jax.ShapeDtypeStruct((2, N, 128), F32),
               jax.ShapeDtypeStruct((N, 1), F32)],
)


def _bcos1_body(agg_ref, y1_ref, dv_ref, wb_ref, w2_ref, z_ref, s_ref, y2_ref):
    a = jnp.concatenate([agg_ref[0], agg_ref[1]], axis=1)
    yy = jnp.concatenate([y1_ref[0], y1_ref[1]], axis=1)
    dv = dv_ref[...]
    z = dv * (a + yy)
    z_ref[...] = z
    rn = jnp.maximum(jnp.sqrt(jnp.sum(z * z, axis=1, keepdims=True)), 1e-12)
    wb = wb_ref[...]
    wn = jnp.maximum(jnp.sqrt(jnp.sum(wb * wb, axis=1)), 1e-12)
    lin = lax.dot_general(z, wb, (((1,), (1,)), ((), ())),
                          preferred_element_type=F32)
    sc = jnp.maximum(lin / (rn * wn[None, :]), 1e-6)
    s_ref[...] = sc
    h = jnp.maximum(lin * sc, 0.0)
    hw = lax.dot_general(h, w2_ref[...], (((1,), (1,)), ((), ())),
                         preferred_element_type=F32)
    y2 = dv * hw
    y2_ref[0] = y2[:, :128]
    y2_ref[1] = y2[:, 128:]


_bcos1 = pl.pallas_call(
    _bcos1_body,
    grid=(GRID,),
    in_specs=[pl.BlockSpec((2, ROWT, 128), lambda i: (0, i, 0)),
              pl.BlockSpec((2, ROWT, 128), lambda i: (0, i, 0)),
              pl.BlockSpec((ROWT, 1), lambda i: (i, 0)),
              pl.BlockSpec((D, D), lambda i: (0, 0)),
              pl.BlockSpec((D, D), lambda i: (0, 0))],
    out_specs=[pl.BlockSpec((ROWT, D), lambda i: (i, 0)),
               pl.BlockSpec((ROWT, D), lambda i: (i, 0)),
               pl.BlockSpec((2, ROWT, 128), lambda i: (0, i, 0))],
    out_shape=[jax.ShapeDtypeStruct((N, D), F32),
               jax.ShapeDtypeStruct((N, D), F32),
               jax.ShapeDtypeStruct((2, N, 128), F32)],
)


def _bcos2_body(agg_ref, y2_ref, dv_ref, wb_ref, z_ref, s_ref, o_ref):
    a = jnp.concatenate([agg_ref[0], agg_ref[1]], axis=1)
    yy = jnp.concatenate([y2_ref[0], y2_ref[1]], axis=1)
    dv = dv_ref[...]
    z = dv * (a + yy)
    z_ref[...] = z
    rn = jnp.maximum(jnp.sqrt(jnp.sum(z * z, axis=1, keepdims=True)), 1e-12)
    wb = wb_ref[...]
    wn = jnp.maximum(jnp.sqrt(jnp.sum(wb * wb, axis=1)), 1e-12)
    lin = lax.dot_general(z, wb, (((1,), (1,)), ((), ())),
                          preferred_element_type=F32)
    sc = jnp.maximum(lin / (rn * wn[None, :]), 1e-6)
    s_ref[...] = sc
    o_ref[...] = lin * sc


_bcos2 = pl.pallas_call(
    _bcos2_body,
    grid=(GRID,),
    in_specs=[pl.BlockSpec((2, ROWT, 128), lambda i: (0, i, 0)),
              pl.BlockSpec((2, ROWT, 128), lambda i: (0, i, 0)),
              pl.BlockSpec((ROWT, 1), lambda i: (i, 0)),
              pl.BlockSpec((128, D), lambda i: (0, 0))],
    out_specs=[pl.BlockSpec((ROWT, D), lambda i: (i, 0)),
               pl.BlockSpec((ROWT, 128), lambda i: (i, 0)),
               pl.BlockSpec((ROWT, 128), lambda i: (i, 0))],
    out_shape=[jax.ShapeDtypeStruct((N, D), F32),
               jax.ShapeDtypeStruct((N, 128), F32),
               jax.ShapeDtypeStruct((N, 128), F32)],
)


def kernel(x, edge_index, W1, W2, Wb1, Wb2):
    src = edge_index[0].astype(jnp.int32)
    dst = edge_index[1].astype(jnp.int32)
    src_p = jnp.concatenate([src, jnp.zeros((EPAD - E,), jnp.int32)])
    dst_p = jnp.concatenate([dst, jnp.full((EPAD - E,), N, jnp.int32)])
    srcs2 = jnp.stack([src_p, src_p + N]).reshape(2, 16, NCHK, CHK)
    dst_agg = dst_p.reshape(16, NCHK, CHK)
    dst_deg = dst_p.reshape(32, NCH_DEG, CHK)
    zrows = jnp.zeros((NROW, 128), F32)
    zrows8 = jnp.zeros((NROW, 8), F32)
    ones8 = jnp.concatenate(
        [jnp.ones((CHK, 1), F32), jnp.zeros((CHK, 7), F32)], axis=1)

    hist8 = _deg()(dst_deg, zrows8, ones8)
    y1, dinv = _mm1s(x, W1, hist8)
    agg1 = _agg()(y1.reshape(2 * N, 128), srcs2, dst_agg, zrows)
    z1, s1, y2 = _bcos1(agg1, y1, dinv, Wb1, W2)
    agg2 = _agg()(y2.reshape(2 * N, 128), srcs2, dst_agg, zrows)
    z2, s2, out = _bcos2(agg2, y2, dinv, Wb2)
    return (out, z1, z2, s1, s2)
